# Initial kernel scaffold; baseline (speedup 1.0000x reference)
import jax
import jax.numpy as jnp
from jax.experimental import pallas as pl


def kernel(inputs, inputs_weight, top_idx, W1, b1, W2, b2):
    raise NotImplementedError("write your pallas kernel here")


# speedup vs baseline: 2.5589x; 2.5589x over previous
import functools

import jax
import jax.numpy as jnp
from jax import lax
from jax.experimental import pallas as pl
from jax.experimental.pallas import tpu as pltpu
from jax.experimental.pallas import tpu_sc as plsc

D_MODEL = 1024
D_FF = 4096
N_TOKENS = 8192
CAPACITY = 2048
N_EXPERTS = 8

NC, NS = 2, 16
ENTRIES = CAPACITY * N_EXPERTS
EPW = ENTRIES // (NC * NS)
FLAT = N_TOKENS * N_EXPERTS
FPW = FLAT // NS


def _sc_hist_body(idx_hbm, out_hbm, idx_v, fidx_v, val_v, zero_v, acc_sh):
    c = lax.axis_index("c")
    s = lax.axis_index("s")
    base = (c * NS + s) * EPW

    pltpu.sync_copy(idx_hbm.at[pl.ds(base, EPW)], idx_v)

    eoff = lax.iota(jnp.int32, 16) & 7
    ones = jnp.ones((16,), jnp.float32)
    zeros = jnp.zeros((16,), jnp.float32)

    def fill(j, carry):
        v = idx_v[pl.ds(j * 16, 16)]
        fidx_v[pl.ds(j * 16, 16)] = v * 8 + eoff
        val_v[pl.ds(j * 16, 16)] = ones
        return carry

    lax.fori_loop(0, EPW // 16, fill, 0)

    def zfill(j, carry):
        zero_v[pl.ds(j * 16, 16)] = zeros
        return carry

    lax.fori_loop(0, FPW // 16, zfill, 0)

    pltpu.sync_copy(zero_v, acc_sh.at[pl.ds(s * FPW, FPW)])
    plsc.subcore_barrier()

    pltpu.sync_copy(val_v, acc_sh.at[fidx_v], add=True)
    plsc.subcore_barrier()

    pltpu.sync_copy(acc_sh.at[pl.ds(s * FPW, FPW)],
                    out_hbm.at[c, pl.ds(s * FPW, FPW)])


@functools.cache
def _sc_hist():
    return pl.kernel(
        _sc_hist_body,
        out_type=jax.ShapeDtypeStruct((NC, FLAT), jnp.float32),
        mesh=plsc.VectorSubcoreMesh(core_axis_name="c", subcore_axis_name="s",
                                    num_cores=NC, num_subcores=NS),
        scratch_types=[
            pltpu.VMEM((EPW,), jnp.int32),
            pltpu.VMEM((EPW,), jnp.int32),
            pltpu.VMEM((EPW,), jnp.float32),
            pltpu.VMEM((FPW,), jnp.float32),
            pltpu.VMEM_SHARED((FLAT,), jnp.float32),
        ],
    )


BT = 256
BF = 1024
NTB = N_TOKENS // BT
NFB = D_FF // BF
RATIO_DENOM = float(CAPACITY * N_EXPERTS * D_FF)


def _tc_ffn_body(x_ref, w1_ref, b1_ref, w2_ref, b2_ref, cnt_ref, iw_ref,
                 out_ref, ratio_ref):
    t = pl.program_id(0)
    f = pl.program_id(1)

    h = jnp.dot(x_ref[...], w1_ref[...],
                preferred_element_type=jnp.float32) + b1_ref[...]
    mask = h > 0.0
    h = jnp.where(mask, h, 0.0)

    cnt = cnt_ref[0] + cnt_ref[1]
    tcount = jnp.sum(cnt, axis=1)

    @pl.when(jnp.logical_and(t == 0, f == 0))
    def _init():
        ratio_ref[...] = jnp.zeros((1, 1), jnp.float32)

    ratio_ref[...] += jnp.sum(
        jnp.sum(mask.astype(jnp.float32), axis=1) * tcount)

    y = jnp.dot(h, w2_ref[...], preferred_element_type=jnp.float32)

    @pl.when(f == 0)
    def _set():
        out_ref[...] = y

    @pl.when(f > 0)
    def _acc():
        out_ref[...] += y

    @pl.when(f == NFB - 1)
    def _finish():
        wsum = jnp.sum(cnt * iw_ref[...], axis=1)
        out_ref[...] = (out_ref[...] + b2_ref[...]) * wsum[:, None]

    @pl.when(jnp.logical_and(t == NTB - 1, f == NFB - 1))
    def _norm():
        ratio_ref[...] = ratio_ref[...] / RATIO_DENOM


_tc_ffn = pl.pallas_call(
    _tc_ffn_body,
    grid=(NTB, NFB),
    in_specs=[
        pl.BlockSpec((BT, D_MODEL), lambda t, f: (t, 0)),
        pl.BlockSpec((D_MODEL, BF), lambda t, f: (0, f)),
        pl.BlockSpec((1, BF), lambda t, f: (0, f)),
        pl.BlockSpec((BF, D_MODEL), lambda t, f: (f, 0)),
        pl.BlockSpec((1, D_MODEL), lambda t, f: (0, 0)),
        pl.BlockSpec((NC, BT, N_EXPERTS), lambda t, f: (0, t, 0)),
        pl.BlockSpec((BT, N_EXPERTS), lambda t, f: (t, 0)),
    ],
    out_specs=[
        pl.BlockSpec((BT, D_MODEL), lambda t, f: (t, 0)),
        pl.BlockSpec((1, 1), lambda t, f: (0, 0)),
    ],
    out_shape=[
        jax.ShapeDtypeStruct((N_TOKENS, D_MODEL), jnp.float32),
        jax.ShapeDtypeStruct((1, 1), jnp.float32),
    ],
    compiler_params=pltpu.CompilerParams(
        dimension_semantics=("arbitrary", "arbitrary")),
)


def kernel(inputs, inputs_weight, top_idx, W1, b1, W2, b2):
    top_flat = top_idx.astype(jnp.int32).reshape(-1)
    counts = _sc_hist()(top_flat)
    counts3 = counts.reshape(NC, N_TOKENS, N_EXPERTS)
    out, ratio = _tc_ffn(inputs, W1, b1.reshape(1, -1), W2,
                         b2.reshape(1, -1), counts3, inputs_weight)
    return out, ratio[0, 0]


# speedup vs baseline: 3.0202x; 1.1803x over previous
import functools

import jax
import jax.numpy as jnp
from jax import lax
from jax.experimental import pallas as pl
from jax.experimental.pallas import tpu as pltpu
from jax.experimental.pallas import tpu_sc as plsc

D_MODEL = 1024
D_FF = 4096
N_TOKENS = 8192
CAPACITY = 2048
N_EXPERTS = 8

NC, NS = 2, 16
ENTRIES = CAPACITY * N_EXPERTS
EPW = ENTRIES // (NC * NS)
FLAT = N_TOKENS * N_EXPERTS
FPW = FLAT // NS


def _sc_hist_body(idx_hbm, out_hbm, idx_v, fidx_v, val_v, zero_v, acc_sh):
    c = lax.axis_index("c")
    s = lax.axis_index("s")
    base = (c * NS + s) * EPW

    pltpu.sync_copy(idx_hbm.at[pl.ds(base, EPW)], idx_v)

    eoff = lax.iota(jnp.int32, 16) & 7
    ones = jnp.ones((16,), jnp.float32)
    zeros = jnp.zeros((16,), jnp.float32)

    def fill(j, carry):
        v = idx_v[pl.ds(j * 16, 16)]
        fidx_v[pl.ds(j * 16, 16)] = v * 8 + eoff
        val_v[pl.ds(j * 16, 16)] = ones
        return carry

    lax.fori_loop(0, EPW // 16, fill, 0)

    def zfill(j, carry):
        zero_v[pl.ds(j * 16, 16)] = zeros
        return carry

    lax.fori_loop(0, FPW // 16, zfill, 0)

    pltpu.sync_copy(zero_v, acc_sh.at[pl.ds(s * FPW, FPW)])
    plsc.subcore_barrier()

    pltpu.sync_copy(val_v, acc_sh.at[fidx_v], add=True)
    plsc.subcore_barrier()

    pltpu.sync_copy(acc_sh.at[pl.ds(s * FPW, FPW)],
                    out_hbm.at[c, pl.ds(s * FPW, FPW)])


@functools.cache
def _sc_hist():
    return pl.kernel(
        _sc_hist_body,
        out_type=jax.ShapeDtypeStruct((NC, FLAT), jnp.float32),
        mesh=plsc.VectorSubcoreMesh(core_axis_name="c", subcore_axis_name="s",
                                    num_cores=NC, num_subcores=NS),
        scratch_types=[
            pltpu.VMEM((EPW,), jnp.int32),
            pltpu.VMEM((EPW,), jnp.int32),
            pltpu.VMEM((EPW,), jnp.float32),
            pltpu.VMEM((FPW,), jnp.float32),
            pltpu.VMEM_SHARED((FLAT,), jnp.float32),
        ],
    )


BT = 256
BF = 1024
NTB = N_TOKENS // BT
NFB = D_FF // BF
RATIO_DENOM = float(CAPACITY * N_EXPERTS * D_FF)


def _tc_ffn_body(x_ref, w1_ref, b1_ref, w2_ref, b2_ref, cnt_ref, iw_ref,
                 out_ref, ratio_ref):
    t = pl.program_id(0)
    f = pl.program_id(1)

    h = jnp.dot(x_ref[...], w1_ref[...],
                preferred_element_type=jnp.float32) + b1_ref[...]
    mask = h > 0.0
    h = jnp.where(mask, h, 0.0).astype(jnp.bfloat16)

    cnt = cnt_ref[0] + cnt_ref[1]
    tcount = jnp.sum(cnt, axis=1)

    @pl.when(jnp.logical_and(t == 0, f == 0))
    def _init():
        ratio_ref[...] = jnp.zeros((1, 1), jnp.float32)

    ratio_ref[...] += jnp.sum(
        jnp.sum(mask.astype(jnp.float32), axis=1) * tcount)

    y = jnp.dot(h, w2_ref[...], preferred_element_type=jnp.float32)

    @pl.when(f == 0)
    def _set():
        out_ref[...] = y

    @pl.when(f > 0)
    def _acc():
        out_ref[...] += y

    @pl.when(f == NFB - 1)
    def _finish():
        wsum = jnp.sum(cnt * iw_ref[...], axis=1)
        out_ref[...] = (out_ref[...] + b2_ref[...]) * wsum[:, None]

    @pl.when(jnp.logical_and(t == NTB - 1, f == NFB - 1))
    def _norm():
        ratio_ref[...] = ratio_ref[...] / RATIO_DENOM


_tc_ffn = pl.pallas_call(
    _tc_ffn_body,
    grid=(NTB, NFB),
    in_specs=[
        pl.BlockSpec((BT, D_MODEL), lambda t, f: (t, 0)),
        pl.BlockSpec((D_MODEL, BF), lambda t, f: (0, f)),
        pl.BlockSpec((1, BF), lambda t, f: (0, f)),
        pl.BlockSpec((BF, D_MODEL), lambda t, f: (f, 0)),
        pl.BlockSpec((1, D_MODEL), lambda t, f: (0, 0)),
        pl.BlockSpec((NC, BT, N_EXPERTS), lambda t, f: (0, t, 0)),
        pl.BlockSpec((BT, N_EXPERTS), lambda t, f: (t, 0)),
    ],
    out_specs=[
        pl.BlockSpec((BT, D_MODEL), lambda t, f: (t, 0)),
        pl.BlockSpec((1, 1), lambda t, f: (0, 0)),
    ],
    out_shape=[
        jax.ShapeDtypeStruct((N_TOKENS, D_MODEL), jnp.float32),
        jax.ShapeDtypeStruct((1, 1), jnp.float32),
    ],
    compiler_params=pltpu.CompilerParams(
        dimension_semantics=("arbitrary", "arbitrary")),
)


def kernel(inputs, inputs_weight, top_idx, W1, b1, W2, b2):
    top_flat = top_idx.astype(jnp.int32).reshape(-1)
    counts = _sc_hist()(top_flat)
    counts3 = counts.reshape(NC, N_TOKENS, N_EXPERTS)
    out, ratio = _tc_ffn(inputs.astype(jnp.bfloat16), W1.astype(jnp.bfloat16),
                         b1.reshape(1, -1), W2.astype(jnp.bfloat16),
                         b2.reshape(1, -1), counts3, inputs_weight)
    return out, ratio[0, 0]


# speedup vs baseline: 4.3113x; 1.4275x over previous
import functools

import jax
import jax.numpy as jnp
from jax import lax
from jax.experimental import pallas as pl
from jax.experimental.pallas import tpu as pltpu
from jax.experimental.pallas import tpu_sc as plsc

D_MODEL = 1024
D_FF = 4096
N_TOKENS = 8192
CAPACITY = 2048
N_EXPERTS = 8

NC, NS = 2, 16
ENTRIES = CAPACITY * N_EXPERTS
EPW = ENTRIES // (NC * NS)
FLAT = N_TOKENS * N_EXPERTS
FPW = FLAT // NS


def _sc_hist_body(idx_hbm, out_hbm, idx_v, fidx_v, val_v, zero_v, acc_sh):
    c = lax.axis_index("c")
    s = lax.axis_index("s")
    base = (c * NS + s) * EPW

    pltpu.sync_copy(idx_hbm.at[pl.ds(base, EPW)], idx_v)

    eoff = lax.iota(jnp.int32, 16) & 7
    ones = jnp.ones((16,), jnp.float32)
    zeros = jnp.zeros((16,), jnp.float32)

    def fill(j, carry):
        v = idx_v[pl.ds(j * 16, 16)]
        fidx_v[pl.ds(j * 16, 16)] = v * 8 + eoff
        val_v[pl.ds(j * 16, 16)] = ones
        return carry

    lax.fori_loop(0, EPW // 16, fill, 0)

    def zfill(j, carry):
        zero_v[pl.ds(j * 16, 16)] = zeros
        return carry

    lax.fori_loop(0, FPW // 16, zfill, 0)

    pltpu.sync_copy(zero_v, acc_sh.at[pl.ds(s * FPW, FPW)])
    plsc.subcore_barrier()

    pltpu.sync_copy(val_v, acc_sh.at[fidx_v], add=True)
    plsc.subcore_barrier()

    pltpu.sync_copy(acc_sh.at[pl.ds(s * FPW, FPW)],
                    out_hbm.at[c, pl.ds(s * FPW, FPW)])


@functools.cache
def _sc_hist():
    return pl.kernel(
        _sc_hist_body,
        out_type=jax.ShapeDtypeStruct((NC, FLAT), jnp.float32),
        mesh=plsc.VectorSubcoreMesh(core_axis_name="c", subcore_axis_name="s",
                                    num_cores=NC, num_subcores=NS),
        scratch_types=[
            pltpu.VMEM((EPW,), jnp.int32),
            pltpu.VMEM((EPW,), jnp.int32),
            pltpu.VMEM((EPW,), jnp.float32),
            pltpu.VMEM((FPW,), jnp.float32),
            pltpu.VMEM_SHARED((FLAT,), jnp.float32),
        ],
    )


BT = 256
NTB = N_TOKENS // BT
RATIO_DENOM = float(CAPACITY * N_EXPERTS * D_FF)


def _tc_ffn_body(x_ref, w1_ref, b1_ref, w2_ref, b2_ref, cnt_ref, iw_ref,
                 out_ref, ratio_ref):
    t = pl.program_id(0)

    h = jnp.dot(x_ref[...], w1_ref[...],
                preferred_element_type=jnp.float32) + b1_ref[...]
    mask = h > 0.0
    hb = jnp.where(mask, h, 0.0).astype(jnp.bfloat16)

    cnt = cnt_ref[0] + cnt_ref[1]
    tcount = jnp.sum(cnt, axis=1)

    @pl.when(t == 0)
    def _init():
        ratio_ref[...] = jnp.zeros((1, 1), jnp.float32)

    ratio_ref[...] += jnp.sum(
        jnp.sum(mask.astype(jnp.float32), axis=1) * tcount)

    y = jnp.dot(hb, w2_ref[...], preferred_element_type=jnp.float32)
    wsum = jnp.sum(cnt * iw_ref[...], axis=1)
    out_ref[...] = (y + b2_ref[...]) * wsum[:, None]

    @pl.when(t == NTB - 1)
    def _norm():
        ratio_ref[...] = ratio_ref[...] / RATIO_DENOM


_tc_ffn = pl.pallas_call(
    _tc_ffn_body,
    grid=(NTB,),
    in_specs=[
        pl.BlockSpec((BT, D_MODEL), lambda t: (t, 0)),
        pl.BlockSpec((D_MODEL, D_FF), lambda t: (0, 0)),
        pl.BlockSpec((1, D_FF), lambda t: (0, 0)),
        pl.BlockSpec((D_FF, D_MODEL), lambda t: (0, 0)),
        pl.BlockSpec((1, D_MODEL), lambda t: (0, 0)),
        pl.BlockSpec((NC, BT, N_EXPERTS), lambda t: (0, t, 0)),
        pl.BlockSpec((BT, N_EXPERTS), lambda t: (t, 0)),
    ],
    out_specs=[
        pl.BlockSpec((BT, D_MODEL), lambda t: (t, 0)),
        pl.BlockSpec((1, 1), lambda t: (0, 0)),
    ],
    out_shape=[
        jax.ShapeDtypeStruct((N_TOKENS, D_MODEL), jnp.float32),
        jax.ShapeDtypeStruct((1, 1), jnp.float32),
    ],
    compiler_params=pltpu.CompilerParams(
        dimension_semantics=("arbitrary",)),
)


def kernel(inputs, inputs_weight, top_idx, W1, b1, W2, b2):
    top_flat = top_idx.astype(jnp.int32).reshape(-1)
    counts = _sc_hist()(top_flat)
    counts3 = counts.reshape(NC, N_TOKENS, N_EXPERTS)
    out, ratio = _tc_ffn(inputs.astype(jnp.bfloat16), W1.astype(jnp.bfloat16),
                         b1.reshape(1, -1), W2.astype(jnp.bfloat16),
                         b2.reshape(1, -1), counts3, inputs_weight)
    return out, ratio[0, 0]


# speedup vs baseline: 4.7865x; 1.1102x over previous
import functools

import jax
import jax.numpy as jnp
from jax import lax
from jax.experimental import pallas as pl
from jax.experimental.pallas import tpu as pltpu
from jax.experimental.pallas import tpu_sc as plsc

D_MODEL = 1024
D_FF = 4096
N_TOKENS = 8192
CAPACITY = 2048
N_EXPERTS = 8

NC, NS = 2, 16
ENTRIES = CAPACITY * N_EXPERTS
EPW = ENTRIES // (NC * NS)
FLAT = N_TOKENS * N_EXPERTS
FPW = FLAT // NS


def _sc_hist_body(idx_hbm, out_hbm, idx_v, fidx_v, val_v, zero_v, acc_sh):
    c = lax.axis_index("c")
    s = lax.axis_index("s")
    base = (c * NS + s) * EPW

    pltpu.sync_copy(idx_hbm.at[pl.ds(base, EPW)], idx_v)

    eoff = lax.iota(jnp.int32, 16) & 7
    ones = jnp.ones((16,), jnp.float32)
    zeros = jnp.zeros((16,), jnp.float32)

    def fill(j, carry):
        v = idx_v[pl.ds(j * 16, 16)]
        fidx_v[pl.ds(j * 16, 16)] = v * 8 + eoff
        val_v[pl.ds(j * 16, 16)] = ones
        return carry

    lax.fori_loop(0, EPW // 16, fill, 0)

    def zfill(j, carry):
        zero_v[pl.ds(j * 16, 16)] = zeros
        return carry

    lax.fori_loop(0, FPW // 16, zfill, 0)

    pltpu.sync_copy(zero_v, acc_sh.at[pl.ds(s * FPW, FPW)])
    plsc.subcore_barrier()

    pltpu.sync_copy(val_v, acc_sh.at[fidx_v], add=True)
    plsc.subcore_barrier()

    pltpu.sync_copy(acc_sh.at[pl.ds(s * FPW, FPW)],
                    out_hbm.at[c, pl.ds(s * FPW, FPW)])


@functools.cache
def _sc_hist():
    return pl.kernel(
        _sc_hist_body,
        out_type=jax.ShapeDtypeStruct((NC, FLAT), jnp.float32),
        mesh=plsc.VectorSubcoreMesh(core_axis_name="c", subcore_axis_name="s",
                                    num_cores=NC, num_subcores=NS),
        scratch_types=[
            pltpu.VMEM((EPW,), jnp.int32),
            pltpu.VMEM((EPW,), jnp.int32),
            pltpu.VMEM((EPW,), jnp.float32),
            pltpu.VMEM((FPW,), jnp.float32),
            pltpu.VMEM_SHARED((FLAT,), jnp.float32),
        ],
    )


BT = 512
NTB = N_TOKENS // BT
RATIO_DENOM = float(CAPACITY * N_EXPERTS * D_FF)


def _tc_ffn_body(x_ref, w1_ref, b1_ref, w2_ref, b2_ref, cnt_ref, iw_ref,
                 out_ref, ratio_ref):
    t = pl.program_id(0)

    h = jnp.dot(x_ref[...].astype(jnp.bfloat16), w1_ref[...],
                preferred_element_type=jnp.float32) + b1_ref[...]
    mask = h > 0.0
    hb = jnp.where(mask, h, 0.0).astype(jnp.bfloat16)

    cnt = cnt_ref[0] + cnt_ref[1]
    tcount = jnp.sum(cnt, axis=1)

    @pl.when(t == 0)
    def _init():
        ratio_ref[...] = jnp.zeros((1, 1), jnp.float32)

    ratio_ref[...] += jnp.sum(
        jnp.sum(mask.astype(jnp.float32), axis=1) * tcount)

    y = jnp.dot(hb, w2_ref[...], preferred_element_type=jnp.float32)
    wsum = jnp.sum(cnt * iw_ref[...], axis=1)
    out_ref[...] = (y + b2_ref[...]) * wsum[:, None]

    @pl.when(t == NTB - 1)
    def _norm():
        ratio_ref[...] = ratio_ref[...] / RATIO_DENOM


_tc_ffn = pl.pallas_call(
    _tc_ffn_body,
    grid=(NTB,),
    in_specs=[
        pl.BlockSpec((BT, D_MODEL), lambda t: (t, 0)),
        pl.BlockSpec((D_MODEL, D_FF), lambda t: (0, 0)),
        pl.BlockSpec((1, D_FF), lambda t: (0, 0)),
        pl.BlockSpec((D_FF, D_MODEL), lambda t: (0, 0)),
        pl.BlockSpec((1, D_MODEL), lambda t: (0, 0)),
        pl.BlockSpec((NC, BT, N_EXPERTS), lambda t: (0, t, 0)),
        pl.BlockSpec((BT, N_EXPERTS), lambda t: (t, 0)),
    ],
    out_specs=[
        pl.BlockSpec((BT, D_MODEL), lambda t: (t, 0)),
        pl.BlockSpec((1, 1), lambda t: (0, 0)),
    ],
    out_shape=[
        jax.ShapeDtypeStruct((N_TOKENS, D_MODEL), jnp.float32),
        jax.ShapeDtypeStruct((1, 1), jnp.float32),
    ],
    compiler_params=pltpu.CompilerParams(
        dimension_semantics=("arbitrary",)),
)


def kernel(inputs, inputs_weight, top_idx, W1, b1, W2, b2):
    top_flat = top_idx.astype(jnp.int32).reshape(-1)
    counts = _sc_hist()(top_flat)
    counts3 = counts.reshape(NC, N_TOKENS, N_EXPERTS)
    out, ratio = _tc_ffn(inputs, W1.astype(jnp.bfloat16),
                         b1.reshape(1, -1), W2.astype(jnp.bfloat16),
                         b2.reshape(1, -1), counts3, inputs_weight)
    return out, ratio[0, 0]


# speedup vs baseline: 4.9393x; 1.0319x over previous
import functools

import jax
import jax.numpy as jnp
from jax import lax
from jax.experimental import pallas as pl
from jax.experimental.pallas import tpu as pltpu
from jax.experimental.pallas import tpu_sc as plsc

D_MODEL = 1024
D_FF = 4096
N_TOKENS = 8192
CAPACITY = 2048
N_EXPERTS = 8

NC, NS = 2, 16
ENTRIES = CAPACITY * N_EXPERTS
EPW = ENTRIES // (NC * NS)
FLAT = N_TOKENS * N_EXPERTS
FPW = FLAT // NS


def _sc_hist_body(idx_hbm, out_hbm, idx_v, fidx_v, val_v, zero_v, acc_sh):
    c = lax.axis_index("c")
    s = lax.axis_index("s")
    base = (c * NS + s) * EPW

    pltpu.sync_copy(idx_hbm.at[pl.ds(base, EPW)], idx_v)

    eoff = lax.iota(jnp.int32, 16) & 7
    ones = jnp.ones((16,), jnp.float32)
    zeros = jnp.zeros((16,), jnp.float32)

    def fill(j, carry):
        v = idx_v[pl.ds(j * 16, 16)]
        fidx_v[pl.ds(j * 16, 16)] = v * 8 + eoff
        val_v[pl.ds(j * 16, 16)] = ones
        return carry

    lax.fori_loop(0, EPW // 16, fill, 0)

    def zfill(j, carry):
        zero_v[pl.ds(j * 16, 16)] = zeros
        return carry

    lax.fori_loop(0, FPW // 16, zfill, 0)

    pltpu.sync_copy(zero_v, acc_sh.at[pl.ds(s * FPW, FPW)])
    plsc.subcore_barrier()

    pltpu.sync_copy(val_v, acc_sh.at[fidx_v], add=True)
    plsc.subcore_barrier()

    pltpu.sync_copy(acc_sh.at[pl.ds(s * FPW, FPW)],
                    out_hbm.at[c, pl.ds(s * FPW, FPW)])


@functools.cache
def _sc_hist():
    return pl.kernel(
        _sc_hist_body,
        out_type=jax.ShapeDtypeStruct((NC, FLAT), jnp.float32),
        mesh=plsc.VectorSubcoreMesh(core_axis_name="c", subcore_axis_name="s",
                                    num_cores=NC, num_subcores=NS),
        scratch_types=[
            pltpu.VMEM((EPW,), jnp.int32),
            pltpu.VMEM((EPW,), jnp.int32),
            pltpu.VMEM((EPW,), jnp.float32),
            pltpu.VMEM((FPW,), jnp.float32),
            pltpu.VMEM_SHARED((FLAT,), jnp.float32),
        ],
    )


BT = 512
FC = 1024
NFC = D_FF // FC
NTB = N_TOKENS // BT
RATIO_DENOM = float(CAPACITY * N_EXPERTS * D_FF)


def _tc_ffn_body(x_ref, w1_ref, b1_ref, w2_ref, b2_ref, cnt_ref, iw_ref,
                 out_ref, ratio_ref):
    t = pl.program_id(0)

    xb = x_ref[...].astype(jnp.bfloat16)
    cnt = cnt_ref[0] + cnt_ref[1]
    tcount = jnp.sum(cnt, axis=1)

    y = None
    nnz_w = None
    for fc in range(NFC):
        h = jnp.dot(xb, w1_ref[:, fc * FC:(fc + 1) * FC],
                    preferred_element_type=jnp.float32)
        h = h + b1_ref[:, fc * FC:(fc + 1) * FC]
        mask = h > 0.0
        hb = jnp.where(mask, h, 0.0).astype(jnp.bfloat16)
        c = jnp.sum(jnp.sum(mask.astype(jnp.float32), axis=1) * tcount)
        nnz_w = c if nnz_w is None else nnz_w + c
        p = jnp.dot(hb, w2_ref[fc * FC:(fc + 1) * FC, :],
                    preferred_element_type=jnp.float32)
        y = p if y is None else y + p

    @pl.when(t == 0)
    def _init():
        ratio_ref[...] = jnp.zeros((1, 1), jnp.float32)

    ratio_ref[...] += nnz_w

    wsum = jnp.sum(cnt * iw_ref[...], axis=1)
    out_ref[...] = (y + b2_ref[...]) * wsum[:, None]

    @pl.when(t == NTB - 1)
    def _norm():
        ratio_ref[...] = ratio_ref[...] / RATIO_DENOM


_tc_ffn = pl.pallas_call(
    _tc_ffn_body,
    grid=(NTB,),
    in_specs=[
        pl.BlockSpec((BT, D_MODEL), lambda t: (t, 0)),
        pl.BlockSpec((D_MODEL, D_FF), lambda t: (0, 0)),
        pl.BlockSpec((1, D_FF), lambda t: (0, 0)),
        pl.BlockSpec((D_FF, D_MODEL), lambda t: (0, 0)),
        pl.BlockSpec((1, D_MODEL), lambda t: (0, 0)),
        pl.BlockSpec((NC, BT, N_EXPERTS), lambda t: (0, t, 0)),
        pl.BlockSpec((BT, N_EXPERTS), lambda t: (t, 0)),
    ],
    out_specs=[
        pl.BlockSpec((BT, D_MODEL), lambda t: (t, 0)),
        pl.BlockSpec((1, 1), lambda t: (0, 0)),
    ],
    out_shape=[
        jax.ShapeDtypeStruct((N_TOKENS, D_MODEL), jnp.float32),
        jax.ShapeDtypeStruct((1, 1), jnp.float32),
    ],
    compiler_params=pltpu.CompilerParams(
        dimension_semantics=("arbitrary",)),
)


def kernel(inputs, inputs_weight, top_idx, W1, b1, W2, b2):
    top_flat = top_idx.astype(jnp.int32).reshape(-1)
    counts = _sc_hist()(top_flat)
    counts3 = counts.reshape(NC, N_TOKENS, N_EXPERTS)
    out, ratio = _tc_ffn(inputs, W1.astype(jnp.bfloat16),
                         b1.reshape(1, -1), W2.astype(jnp.bfloat16),
                         b2.reshape(1, -1), counts3, inputs_weight)
    return out, ratio[0, 0]


# speedup vs baseline: 5.0048x; 1.0133x over previous
import functools

import jax
import jax.numpy as jnp
from jax import lax
from jax.experimental import pallas as pl
from jax.experimental.pallas import tpu as pltpu
from jax.experimental.pallas import tpu_sc as plsc

D_MODEL = 1024
D_FF = 4096
N_TOKENS = 8192
CAPACITY = 2048
N_EXPERTS = 8

NC, NS = 2, 16
ENTRIES = CAPACITY * N_EXPERTS
EPW = ENTRIES // (NC * NS)
FLAT = N_TOKENS * N_EXPERTS
FPW = FLAT // NS


def _sc_hist_body(idx_hbm, out_hbm, idx_v, fidx_v, val_v, zero_v, acc_sh):
    c = lax.axis_index("c")
    s = lax.axis_index("s")
    base = (c * NS + s) * EPW

    pltpu.sync_copy(idx_hbm.at[pl.ds(base, EPW)], idx_v)

    eoff = lax.iota(jnp.int32, 16) & 7
    ones = jnp.ones((16,), jnp.float32)
    zeros = jnp.zeros((16,), jnp.float32)

    def fill(j, carry):
        v = idx_v[pl.ds(j * 16, 16)]
        fidx_v[pl.ds(j * 16, 16)] = v * 8 + eoff
        val_v[pl.ds(j * 16, 16)] = ones
        return carry

    lax.fori_loop(0, EPW // 16, fill, 0)

    def zfill(j, carry):
        zero_v[pl.ds(j * 16, 16)] = zeros
        return carry

    lax.fori_loop(0, FPW // 16, zfill, 0)

    pltpu.sync_copy(zero_v, acc_sh.at[pl.ds(s * FPW, FPW)])
    plsc.subcore_barrier()

    pltpu.sync_copy(val_v, acc_sh.at[fidx_v], add=True)
    plsc.subcore_barrier()

    pltpu.sync_copy(acc_sh.at[pl.ds(s * FPW, FPW)],
                    out_hbm.at[c, pl.ds(s * FPW, FPW)])


@functools.cache
def _sc_hist():
    return pl.kernel(
        _sc_hist_body,
        out_type=jax.ShapeDtypeStruct((NC, FLAT), jnp.float32),
        mesh=plsc.VectorSubcoreMesh(core_axis_name="c", subcore_axis_name="s",
                                    num_cores=NC, num_subcores=NS),
        scratch_types=[
            pltpu.VMEM((EPW,), jnp.int32),
            pltpu.VMEM((EPW,), jnp.int32),
            pltpu.VMEM((EPW,), jnp.float32),
            pltpu.VMEM((FPW,), jnp.float32),
            pltpu.VMEM_SHARED((FLAT,), jnp.float32),
        ],
    )


BT = 1024
FC = 1024
NFC = D_FF // FC
NTB = N_TOKENS // BT
RATIO_DENOM = float(CAPACITY * N_EXPERTS * D_FF)


def _tc_ffn_body(x_ref, w1_ref, b1_ref, w2_ref, b2_ref, cnt_ref, iw_ref,
                 out_ref, ratio_ref):
    t = pl.program_id(0)

    xb = x_ref[...].astype(jnp.bfloat16)
    cnt = cnt_ref[0] + cnt_ref[1]
    tcount = jnp.sum(cnt, axis=1)

    y = None
    nnz_w = None
    for fc in range(NFC):
        h = jnp.dot(xb, w1_ref[:, fc * FC:(fc + 1) * FC],
                    preferred_element_type=jnp.float32)
        h = h + b1_ref[:, fc * FC:(fc + 1) * FC]
        mask = h > 0.0
        hb = jnp.where(mask, h, 0.0).astype(jnp.bfloat16)
        c = jnp.sum(jnp.sum(mask.astype(jnp.float32), axis=1) * tcount)
        nnz_w = c if nnz_w is None else nnz_w + c
        p = jnp.dot(hb, w2_ref[fc * FC:(fc + 1) * FC, :],
                    preferred_element_type=jnp.float32)
        y = p if y is None else y + p

    @pl.when(t == 0)
    def _init():
        ratio_ref[...] = jnp.zeros((1, 1), jnp.float32)

    ratio_ref[...] += nnz_w

    wsum = jnp.sum(cnt * iw_ref[...], axis=1)
    out_ref[...] = (y + b2_ref[...]) * wsum[:, None]

    @pl.when(t == NTB - 1)
    def _norm():
        ratio_ref[...] = ratio_ref[...] / RATIO_DENOM


_tc_ffn = pl.pallas_call(
    _tc_ffn_body,
    grid=(NTB,),
    in_specs=[
        pl.BlockSpec((BT, D_MODEL), lambda t: (t, 0)),
        pl.BlockSpec((D_MODEL, D_FF), lambda t: (0, 0)),
        pl.BlockSpec((1, D_FF), lambda t: (0, 0)),
        pl.BlockSpec((D_FF, D_MODEL), lambda t: (0, 0)),
        pl.BlockSpec((1, D_MODEL), lambda t: (0, 0)),
        pl.BlockSpec((NC, BT, N_EXPERTS), lambda t: (0, t, 0)),
        pl.BlockSpec((BT, N_EXPERTS), lambda t: (t, 0)),
    ],
    out_specs=[
        pl.BlockSpec((BT, D_MODEL), lambda t: (t, 0)),
        pl.BlockSpec((1, 1), lambda t: (0, 0)),
    ],
    out_shape=[
        jax.ShapeDtypeStruct((N_TOKENS, D_MODEL), jnp.float32),
        jax.ShapeDtypeStruct((1, 1), jnp.float32),
    ],
    compiler_params=pltpu.CompilerParams(
        dimension_semantics=("arbitrary",)),
)


def kernel(inputs, inputs_weight, top_idx, W1, b1, W2, b2):
    top_flat = top_idx.astype(jnp.int32).reshape(-1)
    counts = _sc_hist()(top_flat)
    counts3 = counts.reshape(NC, N_TOKENS, N_EXPERTS)
    out, ratio = _tc_ffn(inputs, W1.astype(jnp.bfloat16),
                         b1.reshape(1, -1), W2.astype(jnp.bfloat16),
                         b2.reshape(1, -1), counts3, inputs_weight)
    return out, ratio[0, 0]


# speedup vs baseline: 5.3184x; 1.0627x over previous
import functools

import jax
import jax.numpy as jnp
from jax import lax
from jax.experimental import pallas as pl
from jax.experimental.pallas import tpu as pltpu
from jax.experimental.pallas import tpu_sc as plsc

D_MODEL = 1024
D_FF = 4096
N_TOKENS = 8192
CAPACITY = 2048
N_EXPERTS = 8

NC, NS = 2, 16
ENTRIES = CAPACITY * N_EXPERTS
EPW = ENTRIES // (NC * NS)
FLAT = N_TOKENS * N_EXPERTS
FPW = FLAT // NS


def _sc_hist_body(idx_hbm, out_hbm, idx_v, fidx_v, val_v, zero_v, acc_sh):
    c = lax.axis_index("c")
    s = lax.axis_index("s")
    base = (c * NS + s) * EPW

    pltpu.sync_copy(idx_hbm.at[pl.ds(base, EPW)], idx_v)

    eoff = (lax.iota(jnp.int32, 16) & 7) * N_TOKENS
    ones = jnp.ones((16,), jnp.float32)
    zeros = jnp.zeros((16,), jnp.float32)

    def fill(j, carry):
        v = idx_v[pl.ds(j * 16, 16)]
        fidx_v[pl.ds(j * 16, 16)] = v + eoff
        val_v[pl.ds(j * 16, 16)] = ones
        return carry

    lax.fori_loop(0, EPW // 16, fill, 0)

    def zfill(j, carry):
        zero_v[pl.ds(j * 16, 16)] = zeros
        return carry

    lax.fori_loop(0, FPW // 16, zfill, 0)

    pltpu.sync_copy(zero_v, acc_sh.at[pl.ds(s * FPW, FPW)])
    plsc.subcore_barrier()

    pltpu.sync_copy(val_v, acc_sh.at[fidx_v], add=True)
    plsc.subcore_barrier()

    pltpu.sync_copy(acc_sh.at[pl.ds(s * FPW, FPW)],
                    out_hbm.at[c, s // 2, pl.ds((s % 2) * FPW, FPW)])


@functools.cache
def _sc_hist():
    return pl.kernel(
        _sc_hist_body,
        out_type=jax.ShapeDtypeStruct((NC, N_EXPERTS, N_TOKENS), jnp.float32),
        mesh=plsc.VectorSubcoreMesh(core_axis_name="c", subcore_axis_name="s",
                                    num_cores=NC, num_subcores=NS),
        scratch_types=[
            pltpu.VMEM((EPW,), jnp.int32),
            pltpu.VMEM((EPW,), jnp.int32),
            pltpu.VMEM((EPW,), jnp.float32),
            pltpu.VMEM((FPW,), jnp.float32),
            pltpu.VMEM_SHARED((FLAT,), jnp.float32),
        ],
    )


BT = 1024
FC = 1024
NFC = D_FF // FC
NTB = N_TOKENS // BT
RATIO_DENOM = float(CAPACITY * N_EXPERTS * D_FF)


def _tc_ffn_body(x_ref, w1_ref, b1_ref, w2_ref, b2_ref, cnt_ref, iw_ref,
                 out_ref, ratio_ref):
    t = pl.program_id(0)

    xb = x_ref[...].astype(jnp.bfloat16)
    cnt = cnt_ref[0] + cnt_ref[1]
    tcount = jnp.sum(cnt, axis=0)

    y = None
    nnz_w = None
    for fc in range(NFC):
        h = jnp.dot(xb, w1_ref[:, fc * FC:(fc + 1) * FC],
                    preferred_element_type=jnp.float32)
        h = h + b1_ref[:, fc * FC:(fc + 1) * FC]
        mask = h > 0.0
        hb = jnp.where(mask, h, 0.0).astype(jnp.bfloat16)
        nnz = jnp.sum(mask.astype(jnp.float32), axis=1)
        c = jnp.sum(nnz * tcount)
        nnz_w = c if nnz_w is None else nnz_w + c
        p = jnp.dot(hb, w2_ref[fc * FC:(fc + 1) * FC, :],
                    preferred_element_type=jnp.float32)
        y = p if y is None else y + p

    @pl.when(t == 0)
    def _init():
        ratio_ref[...] = jnp.zeros((1, 1), jnp.float32)

    ratio_ref[...] += nnz_w

    wsum = jnp.sum(cnt * iw_ref[...], axis=0)
    out_ref[...] = (y + b2_ref[...]) * wsum[:, None]

    @pl.when(t == NTB - 1)
    def _norm():
        ratio_ref[...] = ratio_ref[...] / RATIO_DENOM


_tc_ffn = pl.pallas_call(
    _tc_ffn_body,
    grid=(NTB,),
    in_specs=[
        pl.BlockSpec((BT, D_MODEL), lambda t: (t, 0)),
        pl.BlockSpec((D_MODEL, D_FF), lambda t: (0, 0)),
        pl.BlockSpec((1, D_FF), lambda t: (0, 0)),
        pl.BlockSpec((D_FF, D_MODEL), lambda t: (0, 0)),
        pl.BlockSpec((1, D_MODEL), lambda t: (0, 0)),
        pl.BlockSpec((NC, N_EXPERTS, BT), lambda t: (0, 0, t)),
        pl.BlockSpec((N_EXPERTS, BT), lambda t: (0, t)),
    ],
    out_specs=[
        pl.BlockSpec((BT, D_MODEL), lambda t: (t, 0)),
        pl.BlockSpec((1, 1), lambda t: (0, 0)),
    ],
    out_shape=[
        jax.ShapeDtypeStruct((N_TOKENS, D_MODEL), jnp.float32),
        jax.ShapeDtypeStruct((1, 1), jnp.float32),
    ],
    compiler_params=pltpu.CompilerParams(
        dimension_semantics=("arbitrary",)),
)


def kernel(inputs, inputs_weight, top_idx, W1, b1, W2, b2):
    top_flat = top_idx.astype(jnp.int32).reshape(-1)
    counts = _sc_hist()(top_flat)
    out, ratio = _tc_ffn(inputs, W1.astype(jnp.bfloat16),
                         b1.reshape(1, -1), W2.astype(jnp.bfloat16),
                         b2.reshape(1, -1), counts, inputs_weight.T)
    return out, ratio[0, 0]


# speedup vs baseline: 5.4271x; 1.0205x over previous
import functools

import jax
import jax.numpy as jnp
from jax import lax
from jax.experimental import pallas as pl
from jax.experimental.pallas import tpu as pltpu
from jax.experimental.pallas import tpu_sc as plsc

D_MODEL = 1024
D_FF = 4096
N_TOKENS = 8192
CAPACITY = 2048
N_EXPERTS = 8

NC, NS = 2, 16
ENTRIES = CAPACITY * N_EXPERTS
EPW = ENTRIES // (NC * NS)
FLAT = N_TOKENS * N_EXPERTS
FPW = FLAT // NS


def _sc_hist_body(idx_hbm, out_hbm, idx_v, fidx_v, val_v, zero_v, acc_sh):
    c = lax.axis_index("c")
    s = lax.axis_index("s")
    base = (c * NS + s) * EPW

    pltpu.sync_copy(idx_hbm.at[pl.ds(base, EPW)], idx_v)

    eoff = (lax.iota(jnp.int32, 16) & 7) * N_TOKENS
    ones = jnp.ones((16,), jnp.float32)
    zeros = jnp.zeros((16,), jnp.float32)

    def fill(j, carry):
        v = idx_v[pl.ds(j * 16, 16)]
        fidx_v[pl.ds(j * 16, 16)] = v + eoff
        val_v[pl.ds(j * 16, 16)] = ones
        return carry

    lax.fori_loop(0, EPW // 16, fill, 0)

    def zfill(j, carry):
        zero_v[pl.ds(j * 16, 16)] = zeros
        return carry

    lax.fori_loop(0, FPW // 16, zfill, 0)

    pltpu.sync_copy(zero_v, acc_sh.at[pl.ds(s * FPW, FPW)])
    plsc.subcore_barrier()

    pltpu.sync_copy(val_v, acc_sh.at[fidx_v], add=True)
    plsc.subcore_barrier()

    pltpu.sync_copy(acc_sh.at[pl.ds(s * FPW, FPW)],
                    out_hbm.at[c, s // 2, pl.ds((s % 2) * FPW, FPW)])


@functools.cache
def _sc_hist():
    return pl.kernel(
        _sc_hist_body,
        out_type=jax.ShapeDtypeStruct((NC, N_EXPERTS, N_TOKENS), jnp.float32),
        mesh=plsc.VectorSubcoreMesh(core_axis_name="c", subcore_axis_name="s",
                                    num_cores=NC, num_subcores=NS),
        scratch_types=[
            pltpu.VMEM((EPW,), jnp.int32),
            pltpu.VMEM((EPW,), jnp.int32),
            pltpu.VMEM((EPW,), jnp.float32),
            pltpu.VMEM((FPW,), jnp.float32),
            pltpu.VMEM_SHARED((FLAT,), jnp.float32),
        ],
    )


BT = 1024
FC = 2048
NFC = D_FF // FC
NTB = N_TOKENS // BT
RATIO_DENOM = float(CAPACITY * N_EXPERTS * D_FF)


def _tc_ffn_body(x_ref, w1_ref, b1_ref, w2_ref, b2_ref, cnt_ref, iw_ref,
                 out_ref, ratio_ref):
    t = pl.program_id(0)

    xb = x_ref[...].astype(jnp.bfloat16)
    cnt = cnt_ref[0] + cnt_ref[1]
    tcount = jnp.sum(cnt, axis=0)

    y = None
    nnz_w = None
    for fc in range(NFC):
        h = jnp.dot(xb, w1_ref[:, fc * FC:(fc + 1) * FC],
                    preferred_element_type=jnp.float32)
        h = h + b1_ref[:, fc * FC:(fc + 1) * FC]
        mask = h > 0.0
        hb = jnp.where(mask, h, 0.0).astype(jnp.bfloat16)
        nnz = jnp.sum(mask.astype(jnp.float32), axis=1)
        c = jnp.sum(nnz * tcount)
        nnz_w = c if nnz_w is None else nnz_w + c
        p = jnp.dot(hb, w2_ref[fc * FC:(fc + 1) * FC, :],
                    preferred_element_type=jnp.float32)
        y = p if y is None else y + p

    @pl.when(t == 0)
    def _init():
        ratio_ref[...] = jnp.zeros((1, 1), jnp.float32)

    ratio_ref[...] += nnz_w

    wsum = jnp.sum(cnt * iw_ref[...], axis=0)
    out_ref[...] = (y + b2_ref[...]) * wsum[:, None]

    @pl.when(t == NTB - 1)
    def _norm():
        ratio_ref[...] = ratio_ref[...] / RATIO_DENOM


_tc_ffn = pl.pallas_call(
    _tc_ffn_body,
    grid=(NTB,),
    in_specs=[
        pl.BlockSpec((BT, D_MODEL), lambda t: (t, 0)),
        pl.BlockSpec((D_MODEL, D_FF), lambda t: (0, 0)),
        pl.BlockSpec((1, D_FF), lambda t: (0, 0)),
        pl.BlockSpec((D_FF, D_MODEL), lambda t: (0, 0)),
        pl.BlockSpec((1, D_MODEL), lambda t: (0, 0)),
        pl.BlockSpec((NC, N_EXPERTS, BT), lambda t: (0, 0, t)),
        pl.BlockSpec((N_EXPERTS, BT), lambda t: (0, t)),
    ],
    out_specs=[
        pl.BlockSpec((BT, D_MODEL), lambda t: (t, 0)),
        pl.BlockSpec((1, 1), lambda t: (0, 0)),
    ],
    out_shape=[
        jax.ShapeDtypeStruct((N_TOKENS, D_MODEL), jnp.float32),
        jax.ShapeDtypeStruct((1, 1), jnp.float32),
    ],
    compiler_params=pltpu.CompilerParams(
        dimension_semantics=("arbitrary",)),
)


def kernel(inputs, inputs_weight, top_idx, W1, b1, W2, b2):
    top_flat = top_idx.astype(jnp.int32).reshape(-1)
    counts = _sc_hist()(top_flat)
    out, ratio = _tc_ffn(inputs, W1.astype(jnp.bfloat16),
                         b1.reshape(1, -1), W2.astype(jnp.bfloat16),
                         b2.reshape(1, -1), counts, inputs_weight.T)
    return out, ratio[0, 0]
